# baseline (speedup 1.0000x reference)
import functools

import jax
import jax.numpy as jnp
from jax.experimental import pallas as pl
from jax.experimental.pallas import tpu as pltpu

_TOP_K = 8


def _step(x_ref, w_ref, probs_ref, tw_ref, ti_ref, wr_ref, rd_ref, n_experts):
    wr_ref[...] = jax.lax.dot_general(
        x_ref[...], w_ref[...], (((1,), (1,)), ((), ())),
        preferred_element_type=jnp.float32,
    )

    logits = rd_ref[...]
    fcols = jax.lax.broadcasted_iota(
        jnp.int32, logits.shape, 1).astype(jnp.float32)
    sentinel = jnp.float32(n_experts)
    work = logits
    vals = []
    idxs = []
    for _ in range(_TOP_K):
        mk = jnp.max(work, axis=-1, keepdims=True)
        cand = jnp.where(work == mk, fcols, sentinel)
        fik = jnp.min(cand, axis=-1, keepdims=True)
        vals.append(mk)
        idxs.append(fik)
        work = jnp.where(cand == fik, -jnp.inf, work)

    m = vals[0]
    e = jnp.exp(logits - m)
    s = jnp.sum(e, axis=-1, keepdims=True)
    probs_ref[...] = e / s

    lv = jnp.concatenate(vals, axis=-1)
    fti = jnp.concatenate(idxs, axis=-1)
    ev = jnp.exp(lv - m)
    tw_ref[...] = ev / (jnp.sum(ev, axis=-1, keepdims=True) + 1e-6 * s)
    ti_ref[...] = fti.astype(jnp.int32)


def _router_block(x_ref, w_ref, probs_ref, tw_ref, ti_ref, acc_a, acc_b, *,
                  n_experts, n_blocks):
    i = pl.program_id(0)

    @pl.when(i % 2 == 0)
    def _even():
        _step(x_ref, w_ref, probs_ref, tw_ref, ti_ref, acc_a, acc_b,
              n_experts)

    @pl.when(i % 2 == 1)
    def _odd():
        _step(x_ref, w_ref, probs_ref, tw_ref, ti_ref, acc_b, acc_a,
              n_experts)


@functools.partial(jax.jit, static_argnames=("block_t", "interpret"))
def _router(hidden_states, weight, block_t=512, interpret=False):
    t, d = hidden_states.shape
    n_experts = weight.shape[0]
    n_blocks = t // block_t
    return pl.pallas_call(
        functools.partial(_router_block, n_experts=n_experts,
                          n_blocks=n_blocks),
        grid=(n_blocks + 1,),
        in_specs=[
            pl.BlockSpec((block_t, d), lambda i: (jnp.minimum(i, n_blocks - 1), 0)),
            pl.BlockSpec((n_experts, d), lambda i: (0, 0)),
        ],
        out_specs=[
            pl.BlockSpec((block_t, n_experts), lambda i: (jnp.maximum(i - 1, 0), 0)),
            pl.BlockSpec((block_t, _TOP_K), lambda i: (jnp.maximum(i - 1, 0), 0)),
            pl.BlockSpec((block_t, _TOP_K), lambda i: (jnp.maximum(i - 1, 0), 0)),
        ],
        out_shape=[
            jax.ShapeDtypeStruct((t, n_experts), jnp.float32),
            jax.ShapeDtypeStruct((t, _TOP_K), hidden_states.dtype),
            jax.ShapeDtypeStruct((t, _TOP_K), jnp.int32),
        ],
        scratch_shapes=[pltpu.VMEM((block_t, n_experts), jnp.float32),
                        pltpu.VMEM((block_t, n_experts), jnp.float32)],
        interpret=interpret,
    )(hidden_states, weight)


def kernel(hidden_states, weight):
    probs, tw, ti = _router(hidden_states, weight, block_t=256)
    return probs, tw, ti


# speedup vs baseline: 1.3102x; 1.3102x over previous
import functools

import jax
import jax.numpy as jnp
from jax.experimental import pallas as pl
from jax.experimental.pallas import tpu as pltpu

_TOP_K = 8


def _step(x_ref, w_ref, probs_ref, tw_ref, ti_ref, wr_ref, rd_ref, n_experts):
    wr_ref[...] = jax.lax.dot_general(
        x_ref[...], w_ref[...], (((1,), (1,)), ((), ())),
        preferred_element_type=jnp.float32,
    )
    _epilogue(rd_ref, probs_ref, tw_ref, ti_ref, n_experts)


def _epilogue(rd_ref, probs_ref, tw_ref, ti_ref, n_experts):
    logits = rd_ref[...]
    fcols = jax.lax.broadcasted_iota(
        jnp.int32, logits.shape, 1).astype(jnp.float32)
    sentinel = jnp.float32(n_experts)
    work = logits
    vals = []
    idxs = []
    for _ in range(_TOP_K):
        mk = jnp.max(work, axis=-1, keepdims=True)
        cand = jnp.where(work == mk, fcols, sentinel)
        fik = jnp.min(cand, axis=-1, keepdims=True)
        vals.append(mk)
        idxs.append(fik)
        work = jnp.where(cand == fik, -jnp.inf, work)

    m = vals[0]
    e = jnp.exp(logits - m)
    s = jnp.sum(e, axis=-1, keepdims=True)
    probs_ref[...] = e / s

    lv = jnp.concatenate(vals, axis=-1)
    fti = jnp.concatenate(idxs, axis=-1)
    ev = jnp.exp(lv - m)
    tw_ref[...] = ev / (jnp.sum(ev, axis=-1, keepdims=True) + 1e-6 * s)
    ti_ref[...] = fti.astype(jnp.int32)


def _router_block(x_ref, w_ref, probs_ref, tw_ref, ti_ref, acc_a, acc_b, *,
                  n_experts, n_blocks):
    i = pl.program_id(0)

    @pl.when(jnp.logical_and(i % 2 == 0, i < n_blocks))
    def _even():
        _step(x_ref, w_ref, probs_ref, tw_ref, ti_ref, acc_a, acc_b,
              n_experts)

    @pl.when(jnp.logical_and(i % 2 == 1, i < n_blocks))
    def _odd():
        _step(x_ref, w_ref, probs_ref, tw_ref, ti_ref, acc_b, acc_a,
              n_experts)

    last = acc_a if (n_blocks - 1) % 2 == 0 else acc_b

    @pl.when(i == n_blocks)
    def _tail():
        _epilogue(last, probs_ref, tw_ref, ti_ref, n_experts)


@functools.partial(jax.jit, static_argnames=("block_t", "interpret"))
def _router(hidden_states, weight, block_t=512, interpret=False):
    t, d = hidden_states.shape
    n_experts = weight.shape[0]
    n_blocks = t // block_t
    return pl.pallas_call(
        functools.partial(_router_block, n_experts=n_experts,
                          n_blocks=n_blocks),
        grid=(n_blocks + 1,),
        in_specs=[
            pl.BlockSpec((block_t, d), lambda i: (jnp.minimum(i, n_blocks - 1), 0)),
            pl.BlockSpec((n_experts, d), lambda i: (0, 0)),
        ],
        out_specs=[
            pl.BlockSpec((block_t, n_experts), lambda i: (jnp.maximum(i - 1, 0), 0)),
            pl.BlockSpec((block_t, _TOP_K), lambda i: (jnp.maximum(i - 1, 0), 0)),
            pl.BlockSpec((block_t, _TOP_K), lambda i: (jnp.maximum(i - 1, 0), 0)),
        ],
        out_shape=[
            jax.ShapeDtypeStruct((t, n_experts), jnp.float32),
            jax.ShapeDtypeStruct((t, _TOP_K), hidden_states.dtype),
            jax.ShapeDtypeStruct((t, _TOP_K), jnp.int32),
        ],
        scratch_shapes=[pltpu.VMEM((block_t, n_experts), jnp.float32),
                        pltpu.VMEM((block_t, n_experts), jnp.float32)],
        interpret=interpret,
    )(hidden_states, weight)


def kernel(hidden_states, weight):
    probs, tw, ti = _router(hidden_states, weight, block_t=512)
    return probs, tw, ti


# speedup vs baseline: 1.3326x; 1.0171x over previous
import functools

import jax
import jax.numpy as jnp
from jax.experimental import pallas as pl
from jax.experimental.pallas import tpu as pltpu

_TOP_K = 8


def _step(x_ref, w_ref, probs_ref, tw_ref, ti_ref, wr_ref, rd_ref, n_experts):
    wr_ref[...] = jax.lax.dot_general(
        x_ref[...], w_ref[...], (((1,), (1,)), ((), ())),
        preferred_element_type=jnp.float32,
    )
    _epilogue(rd_ref, probs_ref, tw_ref, ti_ref, n_experts)


def _epilogue(rd_ref, probs_ref, tw_ref, ti_ref, n_experts):
    logits = rd_ref[...]
    fcols = jax.lax.broadcasted_iota(
        jnp.int32, logits.shape, 1).astype(jnp.float32)
    sentinel = jnp.float32(n_experts)
    work = logits
    vals = []
    idxs = []
    for _ in range(_TOP_K):
        mk = jnp.max(work, axis=-1, keepdims=True)
        cand = jnp.where(work == mk, fcols, sentinel)
        fik = jnp.min(cand, axis=-1, keepdims=True)
        vals.append(mk)
        idxs.append(fik)
        work = jnp.where(cand == fik, -jnp.inf, work)

    m = vals[0]
    e = jnp.exp(logits - m)
    s = jnp.sum(e, axis=-1, keepdims=True)
    probs_ref[...] = e / s

    lv = jnp.concatenate(vals, axis=-1)
    fti = jnp.concatenate(idxs, axis=-1)
    ev = jnp.exp(lv - m)
    tw_ref[...] = ev / (jnp.sum(ev, axis=-1, keepdims=True) + 1e-6 * s)
    ti_ref[...] = fti.astype(jnp.int32)


_NBUF = 4
_LOOKAHEAD = 3


def _x_copy(x_hbm, xbuf, sems, blk, block_t):
    slot = blk % _NBUF
    return pltpu.make_async_copy(
        x_hbm.at[pl.ds(blk * block_t, block_t), :],
        xbuf.at[slot],
        sems.at[slot],
    )


def _router_block(x_hbm, w_ref, probs_ref, tw_ref, ti_ref, xbuf, acc_a, acc_b,
                  sems, *, n_experts, n_blocks, block_t):
    i = pl.program_id(0)

    @pl.when(i == 0)
    def _prime():
        for b in range(_LOOKAHEAD):
            _x_copy(x_hbm, xbuf, sems, b, block_t).start()

    @pl.when(i + _LOOKAHEAD < n_blocks)
    def _ahead():
        _x_copy(x_hbm, xbuf, sems, i + _LOOKAHEAD, block_t).start()

    @pl.when(jnp.logical_and(i % 2 == 0, i < n_blocks))
    def _even():
        _x_copy(x_hbm, xbuf, sems, i, block_t).wait()
        _step(xbuf.at[i % _NBUF], w_ref, probs_ref, tw_ref, ti_ref, acc_a,
              acc_b, n_experts)

    @pl.when(jnp.logical_and(i % 2 == 1, i < n_blocks))
    def _odd():
        _x_copy(x_hbm, xbuf, sems, i, block_t).wait()
        _step(xbuf.at[i % _NBUF], w_ref, probs_ref, tw_ref, ti_ref, acc_b,
              acc_a, n_experts)

    last = acc_a if (n_blocks - 1) % 2 == 0 else acc_b

    @pl.when(i == n_blocks)
    def _tail():
        _epilogue(last, probs_ref, tw_ref, ti_ref, n_experts)


@functools.partial(jax.jit, static_argnames=("block_t", "interpret"))
def _router(hidden_states, weight, block_t=512, interpret=False):
    t, d = hidden_states.shape
    n_experts = weight.shape[0]
    n_blocks = t // block_t
    return pl.pallas_call(
        functools.partial(_router_block, n_experts=n_experts,
                          n_blocks=n_blocks, block_t=block_t),
        grid=(n_blocks + 1,),
        in_specs=[
            pl.BlockSpec(memory_space=pl.ANY),
            pl.BlockSpec((n_experts, d), lambda i: (0, 0)),
        ],
        out_specs=[
            pl.BlockSpec((block_t, n_experts), lambda i: (jnp.maximum(i - 1, 0), 0)),
            pl.BlockSpec((block_t, _TOP_K), lambda i: (jnp.maximum(i - 1, 0), 0)),
            pl.BlockSpec((block_t, _TOP_K), lambda i: (jnp.maximum(i - 1, 0), 0)),
        ],
        out_shape=[
            jax.ShapeDtypeStruct((t, n_experts), jnp.float32),
            jax.ShapeDtypeStruct((t, _TOP_K), hidden_states.dtype),
            jax.ShapeDtypeStruct((t, _TOP_K), jnp.int32),
        ],
        scratch_shapes=[pltpu.VMEM((_NBUF, block_t, d), jnp.float32),
                        pltpu.VMEM((block_t, n_experts), jnp.float32),
                        pltpu.VMEM((block_t, n_experts), jnp.float32),
                        pltpu.SemaphoreType.DMA((_NBUF,))],
        interpret=interpret,
    )(hidden_states, weight)


def kernel(hidden_states, weight):
    probs, tw, ti = _router(hidden_states, weight, block_t=512)
    return probs, tw, ti


# speedup vs baseline: 1.3340x; 1.0010x over previous
import functools

import jax
import jax.numpy as jnp
from jax.experimental import pallas as pl
from jax.experimental.pallas import tpu as pltpu

_TOP_K = 8


def _step(x_ref, w_ref, probs_ref, tw_ref, ti_ref, wr_ref, rd_ref, n_experts):
    wr_ref[...] = jax.lax.dot_general(
        x_ref[...], w_ref[...], (((1,), (1,)), ((), ())),
        preferred_element_type=jnp.float32,
    )
    _epilogue(rd_ref, probs_ref, tw_ref, ti_ref, n_experts)


def _epilogue(rd_ref, probs_ref, tw_ref, ti_ref, n_experts):
    logits = rd_ref[...]
    fcols = jax.lax.broadcasted_iota(
        jnp.int32, logits.shape, 1).astype(jnp.float32)
    sentinel = jnp.float32(n_experts)
    work = logits
    vals = []
    idxs = []
    for _ in range(_TOP_K):
        mk = jnp.max(work, axis=-1, keepdims=True)
        cand = jnp.where(work == mk, fcols, sentinel)
        fik = jnp.min(cand, axis=-1, keepdims=True)
        vals.append(mk)
        idxs.append(fik)
        work = jnp.where(cand == fik, -jnp.inf, work)

    m = vals[0]
    e = jnp.exp(logits - m)
    s = jnp.sum(e, axis=-1, keepdims=True)
    probs_ref[...] = e / s

    lv = jnp.concatenate(vals, axis=-1)
    fti = jnp.concatenate(idxs, axis=-1)
    ev = jnp.exp(lv - m)
    tw_ref[...] = ev / (jnp.sum(ev, axis=-1, keepdims=True) + 1e-6 * s)
    ti_ref[...] = fti.astype(jnp.int32)


_NBUF = 6
_LOOKAHEAD = 5


def _x_copy(x_hbm, xbuf, sems, blk, block_t):
    slot = blk % _NBUF
    return pltpu.make_async_copy(
        x_hbm.at[pl.ds(blk * block_t, block_t), :],
        xbuf.at[slot],
        sems.at[slot],
    )


def _router_block(x_hbm, w_ref, probs_ref, tw_ref, ti_ref, xbuf, acc_a, acc_b,
                  sems, *, n_experts, n_blocks, block_t):
    i = pl.program_id(0)

    @pl.when(i == 0)
    def _prime():
        for b in range(_LOOKAHEAD):
            _x_copy(x_hbm, xbuf, sems, b, block_t).start()

    @pl.when(i + _LOOKAHEAD < n_blocks)
    def _ahead():
        _x_copy(x_hbm, xbuf, sems, i + _LOOKAHEAD, block_t).start()

    @pl.when(jnp.logical_and(i % 2 == 0, i < n_blocks))
    def _even():
        _x_copy(x_hbm, xbuf, sems, i, block_t).wait()
        _step(xbuf.at[i % _NBUF], w_ref, probs_ref, tw_ref, ti_ref, acc_a,
              acc_b, n_experts)

    @pl.when(jnp.logical_and(i % 2 == 1, i < n_blocks))
    def _odd():
        _x_copy(x_hbm, xbuf, sems, i, block_t).wait()
        _step(xbuf.at[i % _NBUF], w_ref, probs_ref, tw_ref, ti_ref, acc_b,
              acc_a, n_experts)

    last = acc_a if (n_blocks - 1) % 2 == 0 else acc_b

    @pl.when(i == n_blocks)
    def _tail():
        _epilogue(last, probs_ref, tw_ref, ti_ref, n_experts)


@functools.partial(jax.jit, static_argnames=("block_t", "interpret"))
def _router(hidden_states, weight, block_t=512, interpret=False):
    t, d = hidden_states.shape
    n_experts = weight.shape[0]
    n_blocks = t // block_t
    return pl.pallas_call(
        functools.partial(_router_block, n_experts=n_experts,
                          n_blocks=n_blocks, block_t=block_t),
        grid=(n_blocks + 1,),
        in_specs=[
            pl.BlockSpec(memory_space=pl.ANY),
            pl.BlockSpec((n_experts, d), lambda i: (0, 0)),
        ],
        out_specs=[
            pl.BlockSpec((block_t, n_experts), lambda i: (jnp.maximum(i - 1, 0), 0)),
            pl.BlockSpec((block_t, _TOP_K), lambda i: (jnp.maximum(i - 1, 0), 0)),
            pl.BlockSpec((block_t, _TOP_K), lambda i: (jnp.maximum(i - 1, 0), 0)),
        ],
        out_shape=[
            jax.ShapeDtypeStruct((t, n_experts), jnp.float32),
            jax.ShapeDtypeStruct((t, _TOP_K), hidden_states.dtype),
            jax.ShapeDtypeStruct((t, _TOP_K), jnp.int32),
        ],
        scratch_shapes=[pltpu.VMEM((_NBUF, block_t, d), jnp.float32),
                        pltpu.VMEM((block_t, n_experts), jnp.float32),
                        pltpu.VMEM((block_t, n_experts), jnp.float32),
                        pltpu.SemaphoreType.DMA((_NBUF,))],
        interpret=interpret,
    )(hidden_states, weight)


def kernel(hidden_states, weight):
    probs, tw, ti = _router(hidden_states, weight, block_t=512)
    return probs, tw, ti
